# Initial kernel scaffold; baseline (speedup 1.0000x reference)
import jax
import jax.numpy as jnp
from jax.experimental import pallas as pl


def kernel(x, hours, days, pe, hour_encoding, day_encoding):
    raise NotImplementedError("write your pallas kernel here")


# speedup vs baseline: 1.1617x; 1.1617x over previous
import jax
import jax.numpy as jnp
from jax import lax
from jax.experimental import pallas as pl
from jax.experimental.pallas import tpu as pltpu
from jax.experimental.pallas import tpu_sc as plsc

B = 1024
S = 200
D = 128
NC = 2
NS = 16
NW = NC * NS
ROWS_PER_W = B // NW


def _tpe_sc(x, hours, days, pe200, hour_enc, day_enc):
    mesh = plsc.VectorSubcoreMesh(core_axis_name="c", subcore_axis_name="s")

    @pl.kernel(
        out_type=jax.ShapeDtypeStruct((B, S, D), jnp.float32),
        mesh=mesh,
        compiler_params=pltpu.CompilerParams(use_tc_tiling_on_sc=False),
        scratch_types=[
            pltpu.VMEM((S, D), jnp.float32),
            pltpu.VMEM((S, D), jnp.float32),
            pltpu.VMEM((S,), jnp.int32),
            pltpu.VMEM((S,), jnp.int32),
            pltpu.VMEM((S, 32), jnp.float32),
            pltpu.VMEM((S, 32), jnp.float32),
        ],
    )
    def k(x_hbm, h_hbm, d_hbm, pe_hbm, htab_hbm, dtab_hbm, out_hbm,
          pe_v, xv, hv, dv, hr, dr):
        wid = lax.axis_index("s") * NC + lax.axis_index("c")
        pltpu.sync_copy(pe_hbm, pe_v)

        @pl.loop(0, ROWS_PER_W)
        def _(j):
            b = wid * ROWS_PER_W + j
            pltpu.sync_copy(x_hbm.at[b], xv)
            pltpu.sync_copy(h_hbm.at[b], hv)
            pltpu.sync_copy(d_hbm.at[b], dv)
            pltpu.sync_copy(htab_hbm.at[hv], hr)
            pltpu.sync_copy(dtab_hbm.at[dv], dr)

            @pl.loop(0, S)
            def _(t):
                for c in range(8):
                    sl = pl.ds(c * 16, 16)
                    v = xv[t, sl] + pe_v[t, sl]
                    if c < 2:
                        v = v + hr[t, pl.ds(c * 16, 16)]
                    elif c < 4:
                        v = v + dr[t, pl.ds((c - 2) * 16, 16)]
                    xv[t, sl] = v

            pltpu.sync_copy(xv, out_hbm.at[b])

    return k(x, hours, days, pe200, hour_enc, day_enc)


def kernel(x, hours, days, pe, hour_encoding, day_encoding):
    pe200 = pe[0, :S]
    out = _tpe_sc(
        x,
        hours.astype(jnp.int32),
        days.astype(jnp.int32),
        pe200,
        hour_encoding,
        day_encoding,
    )
    return out


# speedup vs baseline: 1.1651x; 1.0029x over previous
import jax
import jax.numpy as jnp
from jax import lax
from jax.experimental import pallas as pl
from jax.experimental.pallas import tpu as pltpu
from jax.experimental.pallas import tpu_sc as plsc

B = 1024
S = 200
D = 128
NC = 2
NS = 16
NW = NC * NS
ROWS_PER_W = B // NW
T = 40
NSLOT = S // T


def _tpe_sc(x, hours, days, pe200, hour_enc, day_enc):
    mesh = plsc.VectorSubcoreMesh(core_axis_name="c", subcore_axis_name="s")

    @pl.kernel(
        out_type=jax.ShapeDtypeStruct((B, S, D), jnp.float32),
        mesh=mesh,
        compiler_params=pltpu.CompilerParams(use_tc_tiling_on_sc=False),
        scratch_types=[
            pltpu.VMEM((S, D), jnp.float32),
            [pltpu.VMEM((T, D), jnp.float32)] * NSLOT,
            [pltpu.VMEM((T,), jnp.int32)] * NSLOT,
            [pltpu.VMEM((T,), jnp.int32)] * NSLOT,
            [pltpu.VMEM((T, 32), jnp.float32)] * NSLOT,
            [pltpu.VMEM((T, 32), jnp.float32)] * NSLOT,
            [pltpu.SemaphoreType.DMA] * NSLOT,
            [pltpu.SemaphoreType.DMA] * NSLOT,
            [pltpu.SemaphoreType.DMA] * NSLOT,
            [pltpu.SemaphoreType.DMA] * NSLOT,
            pltpu.SemaphoreType.DMA,
        ],
    )
    def k(x_hbm, h_hbm, d_hbm, pe_hbm, htab_hbm, dtab_hbm, out_hbm,
          pe_v, xv, hv, dv, hr, dr, xsem, isem, gsem, outsem, psem):
        wid = lax.axis_index("s") * NC + lax.axis_index("c")
        row0 = wid * ROWS_PER_W

        def issue_in(slot, row, s0):
            pltpu.async_copy(x_hbm.at[row, pl.ds(s0, T)], xv[slot], xsem[slot])
            pltpu.async_copy(h_hbm.at[row, pl.ds(s0, T)], hv[slot], isem[slot])
            pltpu.async_copy(d_hbm.at[row, pl.ds(s0, T)], dv[slot], isem[slot])

        def issue_gathers(slot):
            pltpu.make_async_copy(h_hbm.at[0, pl.ds(0, T)], hv[slot], isem[slot]).wait()
            pltpu.make_async_copy(d_hbm.at[0, pl.ds(0, T)], dv[slot], isem[slot]).wait()
            pltpu.async_copy(htab_hbm.at[hv[slot]], hr[slot], gsem[slot])
            pltpu.async_copy(dtab_hbm.at[dv[slot]], dr[slot], gsem[slot])

        def issue_out(slot, row, s0):
            pltpu.async_copy(xv[slot], out_hbm.at[row, pl.ds(s0, T)], outsem[slot])

        def wait_out(slot, row, s0):
            pltpu.make_async_copy(xv[slot], out_hbm.at[row, pl.ds(s0, T)], outsem[slot]).wait()

        pltpu.async_copy(pe_hbm, pe_v, psem)
        for kk in range(NSLOT):
            issue_in(kk, row0, kk * T)
        issue_gathers(0)
        pltpu.make_async_copy(pe_hbm, pe_v, psem).wait()

        @pl.loop(0, ROWS_PER_W)
        def _(g):
            row = row0 + g
            for kk in range(NSLOT):
                s0 = kk * T
                if kk < 2:
                    @pl.when(g >= 1)
                    def _():
                        wait_out(kk + 3, row - 1, (kk + 3) * T)
                        issue_in(kk + 3, row, (kk + 3) * T)
                else:
                    @pl.when(g <= ROWS_PER_W - 2)
                    def _():
                        wait_out(kk - 2, row, (kk - 2) * T)
                        issue_in(kk - 2, row + 1, (kk - 2) * T)

                if kk < NSLOT - 1:
                    issue_gathers(kk + 1)
                else:
                    @pl.when(g <= ROWS_PER_W - 2)
                    def _():
                        issue_gathers(0)

                pltpu.make_async_copy(
                    x_hbm.at[row, pl.ds(s0, T)], xv[kk], xsem[kk]).wait()
                pltpu.make_async_copy(htab_hbm.at[hv[kk]], hr[kk], gsem[kk]).wait()
                pltpu.make_async_copy(dtab_hbm.at[dv[kk]], dr[kk], gsem[kk]).wait()

                @pl.loop(0, T)
                def _(t):
                    for c in range(8):
                        sl = pl.ds(c * 16, 16)
                        v = xv[kk][t, sl] + pe_v[s0 + t, sl]
                        if c < 2:
                            v = v + hr[kk][t, pl.ds(c * 16, 16)]
                        elif c < 4:
                            v = v + dr[kk][t, pl.ds((c - 2) * 16, 16)]
                        xv[kk][t, sl] = v

                issue_out(kk, row, s0)

        last_row = row0 + ROWS_PER_W - 1
        for kk in range(NSLOT):
            wait_out(kk, last_row, kk * T)

    return k(x, hours, days, pe200, hour_enc, day_enc)


def kernel(x, hours, days, pe, hour_encoding, day_encoding):
    pe200 = pe[0, :S]
    out = _tpe_sc(
        x,
        hours.astype(jnp.int32),
        days.astype(jnp.int32),
        pe200,
        hour_encoding,
        day_encoding,
    )
    return out


# speedup vs baseline: 1.7231x; 1.4789x over previous
import jax
import jax.numpy as jnp
from jax import lax
from jax.experimental import pallas as pl
from jax.experimental.pallas import tpu as pltpu
from jax.experimental.pallas import tpu_sc as plsc

B = 1024
S = 200
D = 128
N = B * S
NC = 2
NS = 16
NW = NC * NS
TOK_PER_W = N // NW
T = 32
NBLK = TOK_PER_W // T
NSLOT = 5
NGRP = NBLK // NSLOT


def _tpe_sc(xf, hf, df, pe200, htab_pad, dtab_pad):
    mesh = plsc.VectorSubcoreMesh(core_axis_name="c", subcore_axis_name="s")

    @pl.kernel(
        out_type=jax.ShapeDtypeStruct((N, D), jnp.float32),
        mesh=mesh,
        compiler_params=pltpu.CompilerParams(needs_layout_passes=False),
        scratch_types=[
            pltpu.VMEM((S, D), jnp.float32),
            pltpu.VMEM((24, D), jnp.float32),
            pltpu.VMEM((8, D), jnp.float32),
            pltpu.VMEM((TOK_PER_W,), jnp.int32),
            pltpu.VMEM((TOK_PER_W,), jnp.int32),
            [pltpu.VMEM((T, D), jnp.float32)] * NSLOT,
            pltpu.VMEM_SHARED((NS, NSLOT, T, D), jnp.float32),
            [pltpu.SemaphoreType.DMA] * NSLOT,
            [pltpu.SemaphoreType.DMA] * NSLOT,
            [pltpu.SemaphoreType.DMA] * NSLOT,
            [pltpu.SemaphoreType.DMA] * NSLOT,
            pltpu.SemaphoreType.DMA,
        ],
    )
    def k(x_hbm, h_hbm, d_hbm, pe_hbm, htab_hbm, dtab_hbm, out_hbm,
          pe_v, htab, dtab, hv, dv, xv, sp, s_in, x_in, x_out, s_out, psem):
        sid = lax.axis_index("s")
        wid = sid * NC + lax.axis_index("c")
        tok_base = wid * TOK_PER_W

        def hbm_slice(blk):
            return pl.ds(tok_base + blk * T, T)

        def issue_hbm_in(slot, blk):
            pltpu.async_copy(x_hbm.at[hbm_slice(blk)], sp.at[sid, slot], s_in[slot])

        def wait_hbm_in(slot, blk):
            pltpu.make_async_copy(
                x_hbm.at[hbm_slice(blk)], sp.at[sid, slot], s_in[slot]).wait()

        def issue_cross_in(slot):
            pltpu.async_copy(sp.at[sid, slot], xv[slot], x_in[slot])

        def wait_cross_in(slot):
            pltpu.make_async_copy(sp.at[sid, slot], xv[slot], x_in[slot]).wait()

        def issue_cross_out(slot):
            pltpu.async_copy(xv[slot], sp.at[sid, slot], x_out[slot])

        def wait_cross_out(slot):
            pltpu.make_async_copy(xv[slot], sp.at[sid, slot], x_out[slot]).wait()

        def issue_hbm_out(slot, blk):
            pltpu.async_copy(sp.at[sid, slot], out_hbm.at[hbm_slice(blk)], s_out[slot])

        def wait_hbm_out(slot, blk):
            pltpu.make_async_copy(
                sp.at[sid, slot], out_hbm.at[hbm_slice(blk)], s_out[slot]).wait()

        pltpu.async_copy(pe_hbm, pe_v, psem)
        pltpu.async_copy(htab_hbm, htab, psem)
        pltpu.async_copy(dtab_hbm, dtab, psem)
        pltpu.async_copy(h_hbm.at[pl.ds(tok_base, TOK_PER_W)], hv, psem)
        pltpu.async_copy(d_hbm.at[pl.ds(tok_base, TOK_PER_W)], dv, psem)
        for blk0 in range(3):
            issue_hbm_in(blk0, blk0)
        pltpu.make_async_copy(pe_hbm, pe_v, psem).wait()
        pltpu.make_async_copy(htab_hbm, htab, psem).wait()
        pltpu.make_async_copy(dtab_hbm, dtab, psem).wait()
        pltpu.make_async_copy(h_hbm.at[pl.ds(tok_base, TOK_PER_W)], hv, psem).wait()
        pltpu.make_async_copy(d_hbm.at[pl.ds(tok_base, TOK_PER_W)], dv, psem).wait()
        wait_hbm_in(0, 0)
        issue_cross_in(0)

        @pl.loop(0, NGRP, init_carry=0)
        def _(g, s0g):
            s0 = s0g
            for kk in range(NSLOT):
                blk = g * NSLOT + kk
                k3 = (kk + 3) % NSLOT
                k1 = (kk + 1) % NSLOT

                if kk < 2:
                    @pl.when(g >= 1)
                    def _():
                        wait_hbm_out(k3, blk - 2)

                    issue_hbm_in(k3, blk + 3)
                else:
                    @pl.when(g <= NGRP - 2)
                    def _():
                        wait_hbm_out(k3, blk - 2)
                        issue_hbm_in(k3, blk + 3)

                if kk < NSLOT - 1:
                    wait_hbm_in(k1, blk + 1)
                    issue_cross_in(k1)
                else:
                    @pl.when(g <= NGRP - 2)
                    def _():
                        wait_hbm_in(k1, blk + 1)
                        issue_cross_in(k1)

                wait_cross_in(kk)
                s0k = s0

                @pl.loop(0, T)
                def _(t):
                    s_raw = s0k + t
                    s = jnp.where(s_raw >= S, s_raw - S, s_raw)
                    for c in range(8):
                        sl = pl.ds(c * 16, 16)
                        xv[kk][t, sl] = xv[kk][t, sl] + pe_v[s, sl]

                for t0 in (0, 16):
                    off = blk * T + t0
                    h16 = hv[pl.ds(off, 16)]
                    d16 = dv[pl.ds(off, 16)]
                    tok16 = lax.iota(jnp.int32, 16) + t0

                    @pl.loop(0, 32)
                    def _(c):
                        cvec = jnp.full((16,), c, jnp.int32)
                        hvals = plsc.load_gather(htab, [h16, cvec])
                        plsc.addupdate_scatter(xv[kk], [tok16, cvec], hvals)
                        dvals = plsc.load_gather(dtab, [d16, cvec])
                        plsc.addupdate_scatter(xv[kk], [tok16, cvec + 32], dvals)

                issue_cross_out(kk)

                kp = (kk - 1) % NSLOT
                if kk >= 1:
                    wait_cross_out(kp)
                    issue_hbm_out(kp, blk - 1)
                else:
                    @pl.when(g >= 1)
                    def _():
                        wait_cross_out(kp)
                        issue_hbm_out(kp, blk - 1)

                s0n = s0 + T
                s0 = jnp.where(s0n >= S, s0n - S, s0n)
            return s0

        last = NBLK - 1
        wait_cross_out(NSLOT - 1)
        issue_hbm_out(NSLOT - 1, last)
        for kk in range(NSLOT):
            wait_hbm_out(kk, NBLK - NSLOT + kk)

    return k(xf, hf, df, pe200, htab_pad, dtab_pad)


def kernel(x, hours, days, pe, hour_encoding, day_encoding):
    pe200 = pe[0, :S]
    htab_pad = jnp.zeros((24, D), jnp.float32).at[:, :32].set(hour_encoding)
    dtab_pad = jnp.zeros((8, D), jnp.float32).at[:7, :32].set(day_encoding)
    out = _tpe_sc(
        x.reshape(N, D),
        hours.astype(jnp.int32).reshape(N),
        days.astype(jnp.int32).reshape(N),
        pe200,
        htab_pad,
        dtab_pad,
    )
    return out.reshape(B, S, D)
